# baseline (speedup 1.0000x reference)
import functools

import jax
import jax.numpy as jnp
from jax import lax
from jax.experimental import pallas as pl
from jax.experimental.pallas import tpu as pltpu
from jax.experimental.pallas import tpu_sc as plsc

BATCH = 16384
VOCAB = 32
D = 32
NCOL = 8
OUT_W = NCOL * D
NW = 32
ROWS_W = BATCH // NW
CH = 128
NCH = ROWS_W // CH
CHW = CH * OUT_W
TAB_W = NCOL * VOCAB * D


def _body(c0, c1, c2, c3, c4, c5, c6, c7, wtab, out,
          tab_v, i0, i1, i2, i3, i4, i5, i6, i7,
          buf_a, buf_b, sem_in, sem_a, sem_b):
  cs = (c0, c1, c2, c3, c4, c5, c6, c7)
  ivs = (i0, i1, i2, i3, i4, i5, i6, i7)
  wid = lax.axis_index("s") * 2 + lax.axis_index("c")
  b0 = wid * ROWS_W

  copies = [pltpu.async_copy(wtab, tab_v, sem_in)]
  for j in range(NCOL):
    copies.append(
        pltpu.async_copy(cs[j].at[pl.ds(b0, ROWS_W)], ivs[j], sem_in))
  for cp in copies:
    cp.wait()

  lane = lax.iota(jnp.int32, 16)
  rowbase0 = lane * OUT_W

  def fill(k, buf):
    def group(g, carry):
      dstbase = rowbase0 + g * (16 * OUT_W)
      for j in range(NCOL):
        cvec = ivs[j][pl.ds(pl.multiple_of((k * CH // 16 + g) * 16, 16), 16)]
        srcbase = cvec * D + (j * VOCAB * D)
        for t in range(D):
          vals = plsc.load_gather(tab_v, [srcbase + t])
          plsc.store_scatter(buf, [dstbase + (j * D + t)], vals)
      return carry

    lax.fori_loop(0, CH // 16, group, 0)

  def flush(k, buf, sem):
    return pltpu.async_copy(
        buf, out.at[pl.ds((b0 + k * CH) * OUT_W, CHW)], sem)

  fill(0, buf_a)
  d0 = flush(0, buf_a, sem_a)
  fill(1, buf_b)
  d1 = flush(1, buf_b, sem_b)
  d0.wait()
  fill(2, buf_a)
  d2 = flush(2, buf_a, sem_a)
  d1.wait()
  fill(3, buf_b)
  d3 = flush(3, buf_b, sem_b)
  d2.wait()
  d3.wait()


@jax.jit
def _run(c0, c1, c2, c3, c4, c5, c6, c7, wtab):
  mesh = plsc.VectorSubcoreMesh(core_axis_name="c", subcore_axis_name="s")
  f = pl.kernel(
      _body,
      out_type=jax.ShapeDtypeStruct((BATCH * OUT_W,), jnp.float32),
      mesh=mesh,
      compiler_params=pltpu.CompilerParams(needs_layout_passes=False),
      scratch_types=[
          pltpu.VMEM((TAB_W,), jnp.float32),
      ] + [pltpu.VMEM((ROWS_W,), jnp.int32)] * NCOL + [
          pltpu.VMEM((CHW,), jnp.float32),
          pltpu.VMEM((CHW,), jnp.float32),
          pltpu.SemaphoreType.DMA,
          pltpu.SemaphoreType.DMA,
          pltpu.SemaphoreType.DMA,
      ],
  )
  flat = f(c0, c1, c2, c3, c4, c5, c6, c7, wtab)
  return flat.reshape(BATCH, OUT_W)


def kernel(c0, c1, c2, c3, c4, c5, c6, c7,
           W_c0, W_c1, W_c2, W_c3, W_c4, W_c5, W_c6, W_c7):
  wtab = jnp.concatenate(
      [W_c0, W_c1, W_c2, W_c3, W_c4, W_c5, W_c6, W_c7], axis=0).reshape(TAB_W)
  return _run(c0, c1, c2, c3, c4, c5, c6, c7, wtab)


# speedup vs baseline: 1.7691x; 1.7691x over previous
import functools

import jax
import jax.numpy as jnp
from jax import lax
from jax.experimental import pallas as pl
from jax.experimental.pallas import tpu as pltpu
from jax.experimental.pallas import tpu_sc as plsc

BATCH = 16384
VOCAB = 32
D = 32
NCOL = 8
OUT_W = NCOL * D
NW = 32
ROWS_W = BATCH // NW
CH = 128
NCH = ROWS_W // CH
CHW = CH * OUT_W
TAB_P = D + 1
TAB_W = NCOL * VOCAB * TAB_P
BUF_W = OUT_W + 1


def _body(c0, c1, c2, c3, c4, c5, c6, c7, wtab, out,
          tab_v, i0, i1, i2, i3, i4, i5, i6, i7,
          buf_a, buf_b, sem_in, sem_a, sem_b):
  cs = (c0, c1, c2, c3, c4, c5, c6, c7)
  ivs = (i0, i1, i2, i3, i4, i5, i6, i7)
  wid = lax.axis_index("s") * 2 + lax.axis_index("c")
  b0 = wid * ROWS_W

  copies = [pltpu.async_copy(wtab, tab_v, sem_in)]
  for j in range(NCOL):
    copies.append(
        pltpu.async_copy(cs[j].at[pl.ds(b0, ROWS_W)], ivs[j], sem_in))
  for cp in copies:
    cp.wait()

  lane = lax.iota(jnp.int32, 16)

  def fill(k, buf):
    def group(g, carry):
      rows = lane + g * 16
      for j in range(NCOL):
        cvec = ivs[j][pl.ds(pl.multiple_of((k * CH // 16 + g) * 16, 16), 16)]
        srcbase = cvec * TAB_P + (j * VOCAB * TAB_P)
        for t in range(D):
          vals = plsc.load_gather(tab_v, [srcbase + t])
          cols = jnp.full((16,), j * D + t, jnp.int32)
          plsc.store_scatter(buf, [rows, cols], vals)
      return carry

    lax.fori_loop(0, CH // 16, group, 0)

  def flush(k, buf, sem):
    return pltpu.async_copy(
        buf.at[:, pl.ds(0, OUT_W)], out.at[pl.ds(b0 + k * CH, CH)], sem)

  fill(0, buf_a)
  d0 = flush(0, buf_a, sem_a)
  fill(1, buf_b)
  d1 = flush(1, buf_b, sem_b)
  d0.wait()
  fill(2, buf_a)
  d2 = flush(2, buf_a, sem_a)
  d1.wait()
  fill(3, buf_b)
  d3 = flush(3, buf_b, sem_b)
  d2.wait()
  d3.wait()


@jax.jit
def _run(c0, c1, c2, c3, c4, c5, c6, c7, wtab):
  mesh = plsc.VectorSubcoreMesh(core_axis_name="c", subcore_axis_name="s")
  f = pl.kernel(
      _body,
      out_type=jax.ShapeDtypeStruct((BATCH, OUT_W), jnp.float32),
      mesh=mesh,
      compiler_params=pltpu.CompilerParams(needs_layout_passes=False),
      scratch_types=[
          pltpu.VMEM((TAB_W,), jnp.float32),
      ] + [pltpu.VMEM((ROWS_W,), jnp.int32)] * NCOL + [
          pltpu.VMEM((CH, BUF_W), jnp.float32),
          pltpu.VMEM((CH, BUF_W), jnp.float32),
          pltpu.SemaphoreType.DMA,
          pltpu.SemaphoreType.DMA,
          pltpu.SemaphoreType.DMA,
      ],
  )
  return f(c0, c1, c2, c3, c4, c5, c6, c7, wtab)


def kernel(c0, c1, c2, c3, c4, c5, c6, c7,
           W_c0, W_c1, W_c2, W_c3, W_c4, W_c5, W_c6, W_c7):
  wtab = jnp.pad(
      jnp.concatenate(
          [W_c0, W_c1, W_c2, W_c3, W_c4, W_c5, W_c6, W_c7], axis=0),
      ((0, 0), (0, TAB_P - D))).reshape(TAB_W)
  return _run(c0, c1, c2, c3, c4, c5, c6, c7, wtab)


# speedup vs baseline: 2.6272x; 1.4850x over previous
import functools

import jax
import jax.numpy as jnp
from jax import lax
from jax.experimental import pallas as pl
from jax.experimental.pallas import tpu as pltpu
from jax.experimental.pallas import tpu_sc as plsc

BATCH = 16384
VOCAB = 32
D = 32
NCOL = 8
OUT_W = NCOL * D
NW = 32
NSUBC = 16
ROWS_W = BATCH // NW
CH = 64
NCH = ROWS_W // CH
CHW = CH * OUT_W
SMW = NCOL * CH
IDX_W = NCOL * ROWS_W
TAB_W = NCOL * VOCAB * D


def _body(c0, c1, c2, c3, c4, c5, c6, c7, wtab, out,
          tab_v, ish, sm_a, sm_b, buf_a, buf_b,
          sem_t, sem_i, sem_sa, sem_sb, sem_a, sem_b):
  cs = (c0, c1, c2, c3, c4, c5, c6, c7)
  cid = lax.axis_index("c")
  sid = lax.axis_index("s")
  wid = sid * 2 + cid
  b0 = wid * ROWS_W

  tab_cp = pltpu.async_copy(wtab, tab_v, sem_t)
  icopies = [
      pltpu.async_copy(
          cs[j].at[pl.ds(b0, ROWS_W)],
          ish.at[sid, pl.ds(j * ROWS_W, ROWS_W)], sem_i)
      for j in range(NCOL)
  ]

  def stage_idx(k, sm, sem):
    return [
        pltpu.async_copy(
            ish.at[sid, pl.ds(j * ROWS_W + k * CH, CH)],
            sm.at[pl.ds(j * CH, CH)], sem)
        for j in range(NCOL)
    ]

  for cp in icopies:
    cp.wait()

  sms = (sm_a, sm_b)
  sem_s = (sem_sa, sem_sb)
  bufs = (buf_a, buf_b)
  sem_o = (sem_a, sem_b)

  stages = [None] * (NCH + 1)
  flushes = [None] * NCH
  stages[0] = stage_idx(0, sms[0], sem_s[0])
  tab_cp.wait()

  def fill(k, sm, buf):
    def row(r, carry):
      dst = pl.multiple_of(r * OUT_W, OUT_W)
      for j in range(NCOL):
        c = sm[j * CH + r]
        base = pl.multiple_of(c * D + j * (VOCAB * D), D)
        for h in (0, 16):
          buf[pl.ds(dst + j * D + h, 16)] = tab_v[pl.ds(base + h, 16)]
      return carry

    lax.fori_loop(0, CH, row, 0)

  for k in range(NCH):
    p = k % 2
    if k + 1 < NCH:
      stages[k + 1] = stage_idx(k + 1, sms[1 - p], sem_s[1 - p])
    for cp in stages[k]:
      cp.wait()
    if k >= 2:
      flushes[k - 2].wait()
    fill(k, sms[p], bufs[p])
    flushes[k] = pltpu.async_copy(
        bufs[p], out.at[pl.ds((b0 + k * CH) * OUT_W, CHW)], sem_o[p])

  flushes[NCH - 2].wait()
  flushes[NCH - 1].wait()


@jax.jit
def _run(c0, c1, c2, c3, c4, c5, c6, c7, wtab):
  mesh = plsc.VectorSubcoreMesh(core_axis_name="c", subcore_axis_name="s")
  f = pl.kernel(
      _body,
      out_type=jax.ShapeDtypeStruct((BATCH * OUT_W,), jnp.float32),
      mesh=mesh,
      compiler_params=pltpu.CompilerParams(needs_layout_passes=False),
      scratch_types=[
          pltpu.VMEM((TAB_W,), jnp.float32),
          pltpu.VMEM_SHARED((NSUBC, IDX_W), jnp.int32),
          pltpu.SMEM((SMW,), jnp.int32),
          pltpu.SMEM((SMW,), jnp.int32),
          pltpu.VMEM((CHW,), jnp.float32),
          pltpu.VMEM((CHW,), jnp.float32),
          pltpu.SemaphoreType.DMA,
          pltpu.SemaphoreType.DMA,
          pltpu.SemaphoreType.DMA,
          pltpu.SemaphoreType.DMA,
          pltpu.SemaphoreType.DMA,
          pltpu.SemaphoreType.DMA,
      ],
  )
  flat = f(c0, c1, c2, c3, c4, c5, c6, c7, wtab)
  return flat.reshape(BATCH, OUT_W)


def kernel(c0, c1, c2, c3, c4, c5, c6, c7,
           W_c0, W_c1, W_c2, W_c3, W_c4, W_c5, W_c6, W_c7):
  wtab = jnp.concatenate(
      [W_c0, W_c1, W_c2, W_c3, W_c4, W_c5, W_c6, W_c7], axis=0).reshape(TAB_W)
  return _run(c0, c1, c2, c3, c4, c5, c6, c7, wtab)
